# Initial kernel scaffold; baseline (speedup 1.0000x reference)
import jax
import jax.numpy as jnp
from jax.experimental import pallas as pl


def kernel(node_feat, edge_index, W_neigh, b):
    raise NotImplementedError("write your pallas kernel here")


# speedup vs baseline: 3.4961x; 3.4961x over previous
import functools

import jax
import jax.numpy as jnp
from jax import lax
from jax.experimental import pallas as pl
from jax.experimental.pallas import tpu as pltpu
from jax.experimental.pallas import tpu_sc as plsc

N = 10000
E = 320000
D = 128

NC = 2
NS = 16
CH = 128
KC = 160
EPS = KC * CH
E_PAD = NS * EPS
NPH = 5120
NPC = NPH + 128
RPT = NPC // NS
DUMMY = NPH
NPT = NC * NPH
BAND = 48
NB = NS * BAND


def _sc_aggregate():
    mesh = plsc.VectorSubcoreMesh(
        core_axis_name="c", subcore_axis_name="s", num_cores=NC, num_subcores=NS
    )

    @functools.partial(
        pl.kernel,
        out_type=[
            jax.ShapeDtypeStruct((NC, NPC, D), jnp.float32),
            jax.ShapeDtypeStruct((NC, NB, D), jnp.float32),
        ],
        mesh=mesh,
        scratch_types=[
            pltpu.VMEM((KC, CH), jnp.int32),
            pltpu.VMEM((KC, CH), jnp.int32),
            pltpu.VMEM((CH,), jnp.int32),
            pltpu.VMEM((CH,), jnp.int32),
            pltpu.VMEM((CH, D), jnp.float32),
            pltpu.VMEM((CH, D), jnp.float32),
            pltpu.VMEM_SHARED((NPC, D), jnp.float32),
            pltpu.VMEM_SHARED((NB, D), jnp.float32),
            pltpu.SemaphoreType.DMA,
            pltpu.SemaphoreType.DMA,
        ],
    )
    def body(feat_ext, srcp, dstp, zf, zb, acc_out, deg_out,
             src_idx, dst_idx, oh_idx, dh_idx, rows, ohrows, acc, dacc,
             gsem, osem):
        c = lax.axis_index("c")
        s = lax.axis_index("s")
        base = s * RPT

        pltpu.sync_copy(zf, acc.at[pl.ds(base, RPT)])
        pltpu.sync_copy(zb, dacc.at[pl.ds(s * BAND, BAND)])
        pltpu.sync_copy(srcp.at[s], src_idx)
        pltpu.sync_copy(dstp.at[c, s], dst_idx)
        plsc.subcore_barrier()

        def step(j, carry):
            gcp = pltpu.async_copy(feat_ext.at[src_idx.at[j]], rows, gsem)
            for g in range(CH // 16):
                d16 = dst_idx[j, pl.ds(g * 16, 16)]
                oh_idx[pl.ds(g * 16, 16)] = N + (d16 & 127)
                dh_idx[pl.ds(g * 16, 16)] = (
                    lax.shift_right_logical(d16, 7) + s * BAND
                )
            ocp = pltpu.async_copy(feat_ext.at[oh_idx], ohrows, osem)
            gcp.wait()
            pltpu.sync_copy(rows, acc.at[dst_idx.at[j]], add=True)
            ocp.wait()
            pltpu.sync_copy(ohrows, dacc.at[dh_idx], add=True)
            return carry

        lax.fori_loop(0, KC, step, 0)
        plsc.subcore_barrier()

        pltpu.sync_copy(acc.at[pl.ds(base, RPT)], acc_out.at[c, pl.ds(base, RPT)])
        pltpu.sync_copy(dacc.at[pl.ds(s * BAND, BAND)],
                        deg_out.at[c, pl.ds(s * BAND, BAND)])

    return body


def _tc_body(a_ref, d_ref, x_ref, wt_ref, b_ref, o_ref):
    deg = jnp.sum(d_ref[...], axis=0)[:, None] + 1.0
    h = (a_ref[...] + x_ref[...]) / deg
    o_ref[...] = (
        jnp.dot(h, wt_ref[...], preferred_element_type=jnp.float32) + b_ref[...]
    )


def kernel(node_feat, edge_index, W_neigh, b):
    src = edge_index[0]
    dst = edge_index[1]
    pad = E_PAD - E
    spread = (jnp.arange(pad, dtype=jnp.int32) % 128)
    srcf = jnp.concatenate([src, spread])
    dstf = jnp.concatenate([dst, jnp.asarray(N, jnp.int32) + spread])
    srcp = srcf.reshape(NS, KC, CH)
    dsth = []
    for c in range(NC):
        local = dstf - c * NPH
        inr = (local >= 0) & (local < NPH)
        dloc = jnp.where(inr, local, DUMMY + (srcf % 128))
        dsth.append(dloc.reshape(NS, KC, CH))
    dstp = jnp.stack(dsth)
    feat_ext = jnp.concatenate([node_feat, jnp.eye(D, dtype=jnp.float32)], axis=0)
    zf = jnp.zeros((RPT, D), jnp.float32)
    zb = jnp.zeros((BAND, D), jnp.float32)

    acc_part, deg_part = _sc_aggregate()(feat_ext, srcp, dstp, zf, zb)

    agg = acc_part[:, :NPH, :].reshape(NPT, D)
    degs = (
        deg_part.reshape(NC, NS, BAND * D)[:, :, : NPC]
        [:, :, :NPH]
        .transpose(1, 0, 2)
        .reshape(NS, NPT)
    )
    xp = jnp.concatenate([node_feat, jnp.zeros((NPT - N, D), jnp.float32)], axis=0)
    wt = W_neigh.T
    b2 = b.reshape(1, D)

    BLK = 256
    out = pl.pallas_call(
        _tc_body,
        grid=(NPT // BLK,),
        in_specs=[
            pl.BlockSpec((BLK, D), lambda i: (i, 0)),
            pl.BlockSpec((NS, BLK), lambda i: (0, i)),
            pl.BlockSpec((BLK, D), lambda i: (i, 0)),
            pl.BlockSpec((D, D), lambda i: (0, 0)),
            pl.BlockSpec((1, D), lambda i: (0, 0)),
        ],
        out_specs=pl.BlockSpec((BLK, D), lambda i: (i, 0)),
        out_shape=jax.ShapeDtypeStruct((NPT, D), jnp.float32),
    )(agg, degs, xp, wt, b2)
    return out[:N]
